# baseline (speedup 1.0000x reference)
import jax
import jax.numpy as jnp
from jax import lax
from jax.experimental import pallas as pl
from jax.experimental.pallas import tpu as pltpu
from jax.experimental.pallas import tpu_sc as plsc

N = 320000
D = 128
NUM_SEGMENTS = 1024

NC = 2
NS = 16
NW = NC * NS
ROWS_PER_W = N // NW
NCHUNK = 8
CHUNK = 1264
NBUF = 4
GROUPS = CHUNK // 16
LAST_GROUPS = (ROWS_PER_W - (NCHUNK - 1) * CHUNK) // 16


def _sc_partials_body(feat_hbm, batch_hbm, sums_hbm, cnts_hbm,
                      i0, i1, i2, i3, i4, i5, i6, i7,
                      b0, b1, b2, b3, batch_v, acc_s, acc_c,
                      sem_b, sg0, sg1, sg2, sg3):
    cid = lax.axis_index("c")
    sid = lax.axis_index("s")
    wid = cid * NS + sid
    base = wid * ROWS_PER_W

    idxs = [i0, i1, i2, i3, i4, i5, i6, i7]
    bufs = [b0, b1, b2, b3]
    sems = [sg0, sg1, sg2, sg3]

    scope = jax.named_scope
    batch_cp = pltpu.make_async_copy(
        batch_hbm.at[pl.ds(base, ROWS_PER_W)], batch_v, sem_b)

    lanes = lax.iota(jnp.int32, 16)

    def gather_cp(k, slot):
        return pltpu.make_async_copy(
            feat_hbm.at[idxs[k]], bufs[slot], sems[slot])

    with scope("idx_build"):
      for k in range(NCHUNK):
        def build(g, _, k=k):
            rows = base + k * CHUNK + g * 16 + lanes
            rows = jnp.minimum(rows, N - 1)
            idxs[k][pl.ds(g * 16, 16)] = rows * 8
            return _

        lax.fori_loop(0, GROUPS, build, None)
        if k < NBUF:
            gather_cp(k, k).start()
        if k == 0:
            batch_cp.start()

    with scope("zero_and_batch"):
      def zero(j, _):
          for r in range(8):
              acc_s[r, pl.ds(j * 16, 16)] = jnp.zeros((16,), jnp.float32)
              acc_c[r, pl.ds(j * 16, 16)] = jnp.zeros((16,), jnp.float32)
          return _

      lax.fori_loop(0, 128 // 16, zero, None)
      batch_cp.wait()

    ones = jnp.ones((16,), jnp.float32)
    zeros16 = jnp.zeros((16,), jnp.int32)

    for k in range(NCHUNK):
      with scope(f"chunk{k}"):
        slot = k % NBUF
        gather_cp(k, slot).wait()
        buf = bufs[slot]
        ngroups = GROUPS if k < NCHUNK - 1 else LAST_GROUPS

        def scat(g, _, k=k, buf=buf):
            rows16 = g * 16 + lanes
            vals16 = plsc.load_gather(buf, [rows16, zeros16])
            seg16 = batch_v[pl.ds(k * CHUNK + g * 16, 16)]
            srow16 = lax.shift_right_logical(seg16, 7)
            scol16 = lax.bitwise_and(seg16, 127)
            plsc.addupdate_scatter(acc_s, [srow16, scol16], vals16)
            plsc.addupdate_scatter(acc_c, [srow16, scol16], ones)
            return _

        lax.fori_loop(0, ngroups, scat, None)
        if k + NBUF < NCHUNK:
            gather_cp(k + NBUF, slot).start()

    with scope("publish"):
        pltpu.sync_copy(acc_s, sums_hbm.at[pl.ds(wid * 8, 8)])
        pltpu.sync_copy(acc_c, cnts_hbm.at[pl.ds(wid * 8, 8)])


@jax.jit
def _sc_partials(feat16, batch32):
    mesh = plsc.VectorSubcoreMesh(core_axis_name="c", subcore_axis_name="s")
    return pl.kernel(
        _sc_partials_body,
        out_type=(
            jax.ShapeDtypeStruct((NW * 8, 128), jnp.float32),
            jax.ShapeDtypeStruct((NW * 8, 128), jnp.float32),
        ),
        mesh=mesh,
        compiler_params=pltpu.CompilerParams(
            needs_layout_passes=False, use_tc_tiling_on_sc=False),
        scratch_types=[pltpu.VMEM((CHUNK,), jnp.int32)] * NCHUNK + [
            pltpu.VMEM((CHUNK, 16), jnp.float32),
            pltpu.VMEM((CHUNK, 16), jnp.float32),
            pltpu.VMEM((CHUNK, 16), jnp.float32),
            pltpu.VMEM((CHUNK, 16), jnp.float32),
            pltpu.VMEM((ROWS_PER_W,), jnp.int32),
            pltpu.VMEM((8, 128), jnp.float32),
            pltpu.VMEM((8, 128), jnp.float32),
            pltpu.SemaphoreType.DMA,
            pltpu.SemaphoreType.DMA,
            pltpu.SemaphoreType.DMA,
            pltpu.SemaphoreType.DMA,
            pltpu.SemaphoreType.DMA,
        ],
    )(feat16, batch32)


def _combine_body(s_ref, c_ref, o_ref):
    s = s_ref[...].reshape(NW, 8, 128)
    c = c_ref[...].reshape(NW, 8, 128)
    mean = jnp.sum(s, axis=0) / jnp.maximum(jnp.sum(c, axis=0), 1.0)
    o_ref[...] = mean.reshape(1024)


@jax.jit
def _combine(sums, cnts):
    return pl.pallas_call(
        _combine_body,
        out_shape=jax.ShapeDtypeStruct((NUM_SEGMENTS,), jnp.float32),
    )(sums, cnts)


def kernel(features, batch, is_global):
    del is_global
    feat16 = features.reshape(N * 8, 16)
    batch32 = batch.astype(jnp.int32)
    sums, cnts = _sc_partials(feat16, batch32)
    return _combine(sums, cnts)


# speedup vs baseline: 1.0137x; 1.0137x over previous
import jax
import jax.numpy as jnp
from jax import lax
from jax.experimental import pallas as pl
from jax.experimental.pallas import tpu as pltpu
from jax.experimental.pallas import tpu_sc as plsc

N = 320000
D = 128
NUM_SEGMENTS = 1024

NC = 2
NS = 16
NW = NC * NS
ROWS_PER_W = N // NW
NCHUNK = 8
CHUNK = 1264
NBUF = 3
GROUPS = CHUNK // 16
LAST_GROUPS = (ROWS_PER_W - (NCHUNK - 1) * CHUNK) // 16


def _sc_partials_body(feat_hbm, batch_hbm, sums_hbm, cnts_hbm,
                      i0, i1, i2, i3, i4, i5, i6, i7,
                      b0, b1, b2, batch_v, acc_s, acc_c,
                      sem_b, sg0, sg1, sg2):
    cid = lax.axis_index("c")
    sid = lax.axis_index("s")
    wid = cid * NS + sid
    base = wid * ROWS_PER_W

    idxs = [i0, i1, i2, i3, i4, i5, i6, i7]
    bufs = [b0, b1, b2]
    sems = [sg0, sg1, sg2]

    scope = jax.named_scope
    batch_cp = pltpu.make_async_copy(
        batch_hbm.at[pl.ds(base, ROWS_PER_W)], batch_v, sem_b)

    lanes = lax.iota(jnp.int32, 16)

    def gather_cp(k, slot):
        return pltpu.make_async_copy(
            feat_hbm.at[idxs[k]], bufs[slot], sems[slot])

    with scope("idx_build"):
      for k in range(NCHUNK):
        def build(g, _, k=k):
            rows = base + k * CHUNK + g * 16 + lanes
            rows = jnp.minimum(rows, N - 1)
            idxs[k][pl.ds(g * 16, 16)] = rows * 8
            return _

        lax.fori_loop(0, GROUPS, build, None)
        if k < NBUF:
            gather_cp(k, k).start()
        if k == 0:
            batch_cp.start()

    with scope("zero_and_batch"):
      def zero(j, _):
          for r in range(8):
              acc_s[r, pl.ds(j * 16, 16)] = jnp.zeros((16,), jnp.float32)
              acc_c[r, pl.ds(j * 16, 16)] = jnp.zeros((16,), jnp.float32)
          return _

      lax.fori_loop(0, 128 // 16, zero, None)
      batch_cp.wait()

    ones = jnp.ones((16,), jnp.float32)
    zeros16 = jnp.zeros((16,), jnp.int32)

    for k in range(NCHUNK):
      with scope(f"chunk{k}"):
        slot = k % NBUF
        gather_cp(k, slot).wait()
        buf = bufs[slot]
        ngroups = GROUPS if k < NCHUNK - 1 else LAST_GROUPS

        def scat(g, _, k=k, buf=buf):
            rows16 = g * 16 + lanes
            vals16 = plsc.load_gather(buf, [rows16, zeros16])
            seg16 = batch_v[pl.ds(k * CHUNK + g * 16, 16)]
            srow16 = lax.shift_right_logical(seg16, 7)
            scol16 = lax.bitwise_and(seg16, 127)
            plsc.addupdate_scatter(acc_s, [srow16, scol16], vals16)
            plsc.addupdate_scatter(acc_c, [srow16, scol16], ones)
            return _

        lax.fori_loop(0, ngroups, scat, None)
        if k + NBUF < NCHUNK:
            gather_cp(k + NBUF, slot).start()

    with scope("publish"):
        pltpu.sync_copy(acc_s, sums_hbm.at[pl.ds(wid * 8, 8)])
        pltpu.sync_copy(acc_c, cnts_hbm.at[pl.ds(wid * 8, 8)])


@jax.jit
def _sc_partials(feat16, batch32):
    mesh = plsc.VectorSubcoreMesh(core_axis_name="c", subcore_axis_name="s")
    return pl.kernel(
        _sc_partials_body,
        out_type=(
            jax.ShapeDtypeStruct((NW * 8, 128), jnp.float32),
            jax.ShapeDtypeStruct((NW * 8, 128), jnp.float32),
        ),
        mesh=mesh,
        compiler_params=pltpu.CompilerParams(
            needs_layout_passes=False, use_tc_tiling_on_sc=False),
        scratch_types=[pltpu.VMEM((CHUNK,), jnp.int32)] * NCHUNK + [
            pltpu.VMEM((CHUNK, 16), jnp.float32),
            pltpu.VMEM((CHUNK, 16), jnp.float32),
            pltpu.VMEM((CHUNK, 16), jnp.float32),
            pltpu.VMEM((ROWS_PER_W,), jnp.int32),
            pltpu.VMEM((8, 128), jnp.float32),
            pltpu.VMEM((8, 128), jnp.float32),
            pltpu.SemaphoreType.DMA,
            pltpu.SemaphoreType.DMA,
            pltpu.SemaphoreType.DMA,
            pltpu.SemaphoreType.DMA,
        ],
    )(feat16, batch32)


def _combine_body(s_ref, c_ref, o_ref):
    s = s_ref[...].reshape(NW, 8, 128)
    c = c_ref[...].reshape(NW, 8, 128)
    mean = jnp.sum(s, axis=0) / jnp.maximum(jnp.sum(c, axis=0), 1.0)
    o_ref[...] = mean.reshape(1024)


@jax.jit
def _combine(sums, cnts):
    return pl.pallas_call(
        _combine_body,
        out_shape=jax.ShapeDtypeStruct((NUM_SEGMENTS,), jnp.float32),
    )(sums, cnts)


def kernel(features, batch, is_global):
    del is_global
    feat16 = features.reshape(N * 8, 16)
    batch32 = batch.astype(jnp.int32)
    sums, cnts = _sc_partials(feat16, batch32)
    return _combine(sums, cnts)


# speedup vs baseline: 1.0230x; 1.0092x over previous
import jax
import jax.numpy as jnp
from jax import lax
from jax.experimental import pallas as pl
from jax.experimental.pallas import tpu as pltpu
from jax.experimental.pallas import tpu_sc as plsc

N = 320000
D = 128
NUM_SEGMENTS = 1024

NC = 2
NS = 16
NW = NC * NS
ROWS_PER_W = N // NW
NCHUNK = 8
CHUNK = 1264
NBUF = 3
GROUPS = CHUNK // 16
LAST_GROUPS = (ROWS_PER_W - (NCHUNK - 1) * CHUNK) // 16


def _sc_partials_body(feat_hbm, batch_hbm, sums_hbm, cnts_hbm,
                      i0, i1, i2, i3, i4, i5, i6, i7,
                      b0, b1, b2, batch_v, acc_s, acc_c,
                      sem_b, sg0, sg1, sg2):
    cid = lax.axis_index("c")
    sid = lax.axis_index("s")
    wid = cid * NS + sid
    base = wid * ROWS_PER_W

    idxs = [i0, i1, i2, i3, i4, i5, i6, i7]
    bufs = [b0, b1, b2]
    sems = [sg0, sg1, sg2]

    batch_cp = pltpu.make_async_copy(
        batch_hbm.at[pl.ds(base, ROWS_PER_W)], batch_v, sem_b)
    batch_cp.start()

    lanes = lax.iota(jnp.int32, 16)

    def gather_cp(k, slot):
        return pltpu.make_async_copy(
            feat_hbm.at[idxs[k]], bufs[slot], sems[slot])

    for k in range(NCHUNK):
        def build(g, _, k=k):
            rows = base + k * CHUNK + g * 16 + lanes
            rows = jnp.minimum(rows, N - 1)
            idxs[k][pl.ds(g * 16, 16)] = rows * 8
            return _

        lax.fori_loop(0, GROUPS, build, None)
        if k < NBUF:
            gather_cp(k, k).start()

    def zero(j, _):
        for r in range(8):
            acc_s[r, pl.ds(j * 16, 16)] = jnp.zeros((16,), jnp.float32)
            acc_c[r, pl.ds(j * 16, 16)] = jnp.zeros((16,), jnp.float32)
        return _

    lax.fori_loop(0, 128 // 16, zero, None)
    batch_cp.wait()

    ones = jnp.ones((16,), jnp.float32)

    for k in range(NCHUNK):
        slot = k % NBUF
        gather_cp(k, slot).wait()
        buf = bufs[slot]
        ngroups = GROUPS if k < NCHUNK - 1 else LAST_GROUPS

        def scat(g, _, k=k, buf=buf):
            rows16 = g * 16 + lanes
            zeros16 = jnp.zeros((16,), jnp.int32)
            vals16 = plsc.load_gather(buf, [rows16, zeros16])
            seg16 = batch_v[pl.ds(k * CHUNK + g * 16, 16)]
            srow16 = lax.shift_right_logical(seg16, 7)
            scol16 = lax.bitwise_and(seg16, 127)
            plsc.addupdate_scatter(acc_s, [srow16, scol16], vals16)
            plsc.addupdate_scatter(acc_c, [srow16, scol16], ones)
            return _

        lax.fori_loop(0, ngroups, scat, None)
        if k + NBUF < NCHUNK:
            gather_cp(k + NBUF, slot).start()

    pltpu.sync_copy(acc_s, sums_hbm.at[pl.ds(wid * 8, 8)])
    pltpu.sync_copy(acc_c, cnts_hbm.at[pl.ds(wid * 8, 8)])


@jax.jit
def _sc_partials(feat16, batch32):
    mesh = plsc.VectorSubcoreMesh(core_axis_name="c", subcore_axis_name="s")
    return pl.kernel(
        _sc_partials_body,
        out_type=(
            jax.ShapeDtypeStruct((NW * 8, 128), jnp.float32),
            jax.ShapeDtypeStruct((NW * 8, 128), jnp.float32),
        ),
        mesh=mesh,
        compiler_params=pltpu.CompilerParams(
            needs_layout_passes=False, use_tc_tiling_on_sc=False),
        scratch_types=[pltpu.VMEM((CHUNK,), jnp.int32)] * NCHUNK + [
            pltpu.VMEM((CHUNK, 16), jnp.float32),
            pltpu.VMEM((CHUNK, 16), jnp.float32),
            pltpu.VMEM((CHUNK, 16), jnp.float32),
            pltpu.VMEM((ROWS_PER_W,), jnp.int32),
            pltpu.VMEM((8, 128), jnp.float32),
            pltpu.VMEM((8, 128), jnp.float32),
            pltpu.SemaphoreType.DMA,
            pltpu.SemaphoreType.DMA,
            pltpu.SemaphoreType.DMA,
            pltpu.SemaphoreType.DMA,
        ],
    )(feat16, batch32)


def _combine_body(s_ref, c_ref, o_ref):
    s = s_ref[...].reshape(NW, 8, 128)
    c = c_ref[...].reshape(NW, 8, 128)
    mean = jnp.sum(s, axis=0) / jnp.maximum(jnp.sum(c, axis=0), 1.0)
    o_ref[...] = mean.reshape(1024)


@jax.jit
def _combine(sums, cnts):
    return pl.pallas_call(
        _combine_body,
        out_shape=jax.ShapeDtypeStruct((NUM_SEGMENTS,), jnp.float32),
    )(sums, cnts)


def kernel(features, batch, is_global):
    del is_global
    feat16 = features.reshape(N * 8, 16)
    batch32 = batch.astype(jnp.int32)
    sums, cnts = _sc_partials(feat16, batch32)
    return _combine(sums, cnts)


# speedup vs baseline: 1.0462x; 1.0227x over previous
import jax
import jax.numpy as jnp
from jax import lax
from jax.experimental import pallas as pl
from jax.experimental.pallas import tpu as pltpu
from jax.experimental.pallas import tpu_sc as plsc

N = 320000
D = 128
NUM_SEGMENTS = 1024

NC = 2
NS = 16
NW = NC * NS
ROWS_PER_W = N // NW
NCHUNK = 16
CHUNK = 640
NBUF = 3
GROUPS = CHUNK // 16
LAST_GROUPS = (ROWS_PER_W - (NCHUNK - 1) * CHUNK) // 16


def _sc_partials_body(feat_hbm, batch_hbm, sums_hbm, cnts_hbm,
                      i0, i1, i2, i3, i4, i5, i6, i7,
                      i8, i9, i10, i11, i12, i13, i14, i15,
                      b0, b1, b2, batch_v, acc_s, acc_c,
                      sem_b, sg0, sg1, sg2):
    cid = lax.axis_index("c")
    sid = lax.axis_index("s")
    wid = cid * NS + sid
    base = wid * ROWS_PER_W

    idxs = [i0, i1, i2, i3, i4, i5, i6, i7,
            i8, i9, i10, i11, i12, i13, i14, i15]
    bufs = [b0, b1, b2]
    sems = [sg0, sg1, sg2]

    batch_cp = pltpu.make_async_copy(
        batch_hbm.at[pl.ds(base, ROWS_PER_W)], batch_v, sem_b)
    batch_cp.start()

    lanes = lax.iota(jnp.int32, 16)

    def gather_cp(k, slot):
        return pltpu.make_async_copy(
            feat_hbm.at[idxs[k]], bufs[slot], sems[slot])

    for k in range(NCHUNK):
        def build(g, _, k=k):
            rows = base + k * CHUNK + g * 16 + lanes
            rows = jnp.minimum(rows, N - 1)
            idxs[k][pl.ds(g * 16, 16)] = rows * 8
            return _

        lax.fori_loop(0, GROUPS, build, None)
        if k < NBUF:
            gather_cp(k, k).start()

    def zero(j, _):
        for r in range(8):
            acc_s[r, pl.ds(j * 16, 16)] = jnp.zeros((16,), jnp.float32)
            acc_c[r, pl.ds(j * 16, 16)] = jnp.zeros((16,), jnp.float32)
        return _

    lax.fori_loop(0, 128 // 16, zero, None)
    batch_cp.wait()

    ones = jnp.ones((16,), jnp.float32)

    for k in range(NCHUNK):
        slot = k % NBUF
        gather_cp(k, slot).wait()
        buf = bufs[slot]
        ngroups = GROUPS if k < NCHUNK - 1 else LAST_GROUPS

        def scat(g, _, k=k, buf=buf):
            rows16 = g * 16 + lanes
            zeros16 = jnp.zeros((16,), jnp.int32)
            vals16 = plsc.load_gather(buf, [rows16, zeros16])
            seg16 = batch_v[pl.ds(k * CHUNK + g * 16, 16)]
            srow16 = lax.shift_right_logical(seg16, 7)
            scol16 = lax.bitwise_and(seg16, 127)
            plsc.addupdate_scatter(acc_s, [srow16, scol16], vals16)
            plsc.addupdate_scatter(acc_c, [srow16, scol16], ones)
            return _

        lax.fori_loop(0, ngroups, scat, None)
        if k + NBUF < NCHUNK:
            gather_cp(k + NBUF, slot).start()

    pltpu.sync_copy(acc_s, sums_hbm.at[pl.ds(wid * 8, 8)])
    pltpu.sync_copy(acc_c, cnts_hbm.at[pl.ds(wid * 8, 8)])


@jax.jit
def _sc_partials(feat16, batch32):
    mesh = plsc.VectorSubcoreMesh(core_axis_name="c", subcore_axis_name="s")
    return pl.kernel(
        _sc_partials_body,
        out_type=(
            jax.ShapeDtypeStruct((NW * 8, 128), jnp.float32),
            jax.ShapeDtypeStruct((NW * 8, 128), jnp.float32),
        ),
        mesh=mesh,
        compiler_params=pltpu.CompilerParams(
            needs_layout_passes=False, use_tc_tiling_on_sc=False),
        scratch_types=[pltpu.VMEM((CHUNK,), jnp.int32)] * NCHUNK + [
            pltpu.VMEM((CHUNK, 16), jnp.float32),
            pltpu.VMEM((CHUNK, 16), jnp.float32),
            pltpu.VMEM((CHUNK, 16), jnp.float32),
            pltpu.VMEM((ROWS_PER_W,), jnp.int32),
            pltpu.VMEM((8, 128), jnp.float32),
            pltpu.VMEM((8, 128), jnp.float32),
            pltpu.SemaphoreType.DMA,
            pltpu.SemaphoreType.DMA,
            pltpu.SemaphoreType.DMA,
            pltpu.SemaphoreType.DMA,
        ],
    )(feat16, batch32)


def _combine_body(s_ref, c_ref, o_ref):
    s = s_ref[...].reshape(NW, 8, 128)
    c = c_ref[...].reshape(NW, 8, 128)
    mean = jnp.sum(s, axis=0) / jnp.maximum(jnp.sum(c, axis=0), 1.0)
    o_ref[...] = mean.reshape(1024)


@jax.jit
def _combine(sums, cnts):
    return pl.pallas_call(
        _combine_body,
        out_shape=jax.ShapeDtypeStruct((NUM_SEGMENTS,), jnp.float32),
    )(sums, cnts)


def kernel(features, batch, is_global):
    del is_global
    feat16 = features.reshape(N * 8, 16)
    batch32 = batch.astype(jnp.int32)
    sums, cnts = _sc_partials(feat16, batch32)
    return _combine(sums, cnts)
